# baseline (speedup 1.0000x reference)
import functools

import jax
import jax.numpy as jnp
from jax import lax
from jax.experimental import pallas as pl
from jax.experimental.pallas import tpu as pltpu
from jax.experimental.pallas import tpu_sc as plsc

EMBED_DIM = 64
NUM_VALS = 52
NC = 2
NS = 16
NW = NC * NS
BLK = 128
L = 16


def _table_tc(card, rank52, suit52):
    def body(c_ref, r_ref, s_ref, o_ref):
        h = c_ref[...] + r_ref[...] + s_ref[...]
        mean = jnp.mean(h, axis=-1, keepdims=True)
        var = jnp.mean(jnp.square(h - mean), axis=-1, keepdims=True)
        o_ref[...] = (h - mean) * lax.rsqrt(var + 1e-5)

    return pl.pallas_call(
        body,
        out_shape=jax.ShapeDtypeStruct((NUM_VALS, EMBED_DIM), jnp.float32),
    )(card, rank52, suit52)


@functools.lru_cache(maxsize=None)
def _make_gather_sc(bn: int, num_cards: int):
    assert bn % (NW * BLK) == 0
    blk_per_w = bn // (NW * BLK)
    b_per_w = blk_per_w * BLK
    tok_per_w = b_per_w * num_cards
    n_units = num_cards * blk_per_w

    mesh = plsc.VectorSubcoreMesh(
        core_axis_name="c", subcore_axis_name="s", num_cores=NC, num_subcores=NS
    )

    @functools.partial(
        pl.kernel,
        mesh=mesh,
        out_type=jax.ShapeDtypeStruct((num_cards, EMBED_DIM, bn), jnp.float32),
        scratch_types=[
            pltpu.VMEM((NUM_VALS * EMBED_DIM,), jnp.float32),
            pltpu.VMEM((tok_per_w,), jnp.int32),
            [pltpu.VMEM((EMBED_DIM, BLK), jnp.float32) for _ in range(2)],
            [pltpu.SemaphoreType.DMA for _ in range(2)],
        ],
        compiler_params=pltpu.CompilerParams(
            use_tc_tiling_on_sc=False, needs_layout_passes=False
        ),
    )
    def gather(table_hbm, xf_hbm, out_hbm, tab1, xloc, bufs, wsems):
        sid = lax.axis_index("s")
        wid = sid * NC + lax.axis_index("c")
        base_b = wid * b_per_w

        pltpu.sync_copy(table_hbm, tab1)
        pltpu.sync_copy(xf_hbm.at[pl.ds(wid * tok_per_w, tok_per_w)], xloc)

        i20 = lax.iota(jnp.int32, L) * num_cards

        def do_unit(u, buf, wsem, first):
            c = u % num_cards
            blk = u // num_cards

            @pl.when(jnp.logical_not(first))
            def _():
                pltpu.make_async_copy(
                    buf, out_hbm.at[0, :, pl.ds(0, BLK)], wsem
                ).wait()

            for h in range(BLK // L):
                pos = i20 + ((blk * BLK + h * L) * num_cards + c)
                v = plsc.load_gather(xloc, [pos])
                v64 = v * EMBED_DIM
                for e in range(EMBED_DIM):
                    buf[e, pl.ds(h * L, L)] = plsc.load_gather(
                        tab1, [v64 + e]
                    )
            pltpu.async_copy(
                buf,
                out_hbm.at[c, :, pl.ds(base_b + blk * BLK, BLK)],
                wsem,
            )

        def body(p, carry):
            do_unit(2 * p, bufs[0], wsems[0], p == 0)
            do_unit(2 * p + 1, bufs[1], wsems[1], p == 0)
            return carry

        lax.fori_loop(0, n_units // 2, body, 0, unroll=False)
        for b in range(2):
            pltpu.make_async_copy(
                bufs[b], out_hbm.at[0, :, pl.ds(0, BLK)], wsems[b]
            ).wait()

    return gather


def kernel(x, card, rank, suit):
    bn, num_cards = x.shape
    rank52 = jnp.tile(rank, (NUM_VALS // 13, 1))
    suit52 = jnp.repeat(suit, 13, axis=0)
    table = _table_tc(card, rank52, suit52)
    out_t = _make_gather_sc(bn, num_cards)(table.reshape(-1), x.reshape(-1))
    return jnp.transpose(out_t, (2, 0, 1))


# speedup vs baseline: 2.1454x; 2.1454x over previous
import functools

import jax
import jax.numpy as jnp
from jax import lax
from jax.experimental import pallas as pl
from jax.experimental.pallas import tpu as pltpu
from jax.experimental.pallas import tpu_sc as plsc

EMBED_DIM = 64
NUM_VALS = 52
NPAIR = NUM_VALS * NUM_VALS
NC = 2
NS = 16
NW = NC * NS
L = 16


def _pair_table_tc(card, rank52, suit52):
    def body(c_ref, r_ref, s_ref, o_ref):
        h = c_ref[...] + r_ref[...] + s_ref[...]
        mean = jnp.mean(h, axis=-1, keepdims=True)
        var = jnp.mean(jnp.square(h - mean), axis=-1, keepdims=True)
        t = (h - mean) * lax.rsqrt(var + 1e-5)
        a = jnp.broadcast_to(t[:, None, :], (NUM_VALS, NUM_VALS, EMBED_DIM))
        b = jnp.broadcast_to(t[None, :, :], (NUM_VALS, NUM_VALS, EMBED_DIM))
        o_ref[...] = jnp.concatenate([a, b], axis=-1).reshape(
            NPAIR, 2 * EMBED_DIM
        )

    return pl.pallas_call(
        body,
        out_shape=jax.ShapeDtypeStruct((NPAIR, 2 * EMBED_DIM), jnp.float32),
    )(card, rank52, suit52)


@functools.lru_cache(maxsize=None)
def _make_gather_sc(n_tok: int):
    assert n_tok % (2 * NW) == 0
    tok_per_w = n_tok // NW
    np_w = tok_per_w // 2
    chunk = 128
    while np_w % chunk:
        chunk //= 2
    n_chunks = np_w // chunk
    nbuf = min(3, n_chunks)

    mesh = plsc.VectorSubcoreMesh(
        core_axis_name="c", subcore_axis_name="s", num_cores=NC, num_subcores=NS
    )

    @functools.partial(
        pl.kernel,
        mesh=mesh,
        out_type=jax.ShapeDtypeStruct((n_tok // 2, 2 * EMBED_DIM), jnp.float32),
        scratch_types=[
            pltpu.VMEM((tok_per_w,), jnp.int32),
            pltpu.VMEM((np_w,), jnp.int32),
            [pltpu.VMEM((chunk, 2 * EMBED_DIM), jnp.float32) for _ in range(nbuf)],
            pltpu.VMEM_SHARED((NPAIR, 2 * EMBED_DIM), jnp.float32),
            [pltpu.SemaphoreType.DMA for _ in range(nbuf)],
            [pltpu.SemaphoreType.DMA for _ in range(nbuf)],
        ],
        compiler_params=pltpu.CompilerParams(
            use_tc_tiling_on_sc=False, needs_layout_passes=False
        ),
    )
    def gather(tab2_hbm, x_hbm, out_hbm, xloc, xp, rows, tab2_sh, gsems, ssems):
        sid = lax.axis_index("s")
        wid = sid * NC + lax.axis_index("c")
        base = wid * np_w

        @pl.when(sid == 0)
        def _():
            pltpu.sync_copy(tab2_hbm, tab2_sh)

        pltpu.sync_copy(x_hbm.at[pl.ds(wid * tok_per_w, tok_per_w)], xloc)

        i2 = lax.iota(jnp.int32, L) * 2

        def mk_idx(i, carry):
            base_t = i * (2 * L)
            ev = plsc.load_gather(xloc, [i2 + base_t])
            od = plsc.load_gather(xloc, [i2 + (base_t + 1)])
            xp[pl.ds(i * L, L)] = ev * NUM_VALS + od
            return carry

        lax.fori_loop(0, np_w // L, mk_idx, 0)

        plsc.subcore_barrier()

        def start_gather(c):
            b = c % nbuf
            return pltpu.async_copy(
                tab2_sh.at[xp.at[pl.ds(c * chunk, chunk)]], rows[b], gsems[b]
            )

        def start_scatter(c):
            b = c % nbuf
            return pltpu.async_copy(
                rows[b], out_hbm.at[pl.ds(base + c * chunk, chunk)], ssems[b]
            )

        g_h = {0: start_gather(0)}
        s_h = {}
        s_waited = set()
        for c in range(n_chunks):
            g_h[c].wait()
            if c + 1 < n_chunks:
                prev = c + 1 - nbuf
                if prev >= 0:
                    s_h[prev].wait()
                    s_waited.add(prev)
                g_h[c + 1] = start_gather(c + 1)
            s_h[c] = start_scatter(c)
        for c in range(n_chunks):
            if c not in s_waited:
                s_h[c].wait()

    return gather


def kernel(x, card, rank, suit):
    bn, num_cards = x.shape
    rank52 = jnp.tile(rank, (NUM_VALS // 13, 1))
    suit52 = jnp.repeat(suit, 13, axis=0)
    table2 = _pair_table_tc(card, rank52, suit52)
    out2 = _make_gather_sc(bn * num_cards)(table2, x.reshape(-1))
    return out2.reshape(bn, num_cards, EMBED_DIM)


# speedup vs baseline: 2.2162x; 1.0330x over previous
import functools

import jax
import jax.numpy as jnp
from jax import lax
from jax.experimental import pallas as pl
from jax.experimental.pallas import tpu as pltpu
from jax.experimental.pallas import tpu_sc as plsc

EMBED_DIM = 64
NUM_VALS = 52
NPAIR = NUM_VALS * NUM_VALS
NC = 2
NS = 16
NW = NC * NS
L = 16


def _pair_table_tc(card, rank52, suit52):
    def body(c_ref, r_ref, s_ref, o_ref):
        h = c_ref[...] + r_ref[...] + s_ref[...]
        mean = jnp.mean(h, axis=-1, keepdims=True)
        var = jnp.mean(jnp.square(h - mean), axis=-1, keepdims=True)
        t = (h - mean) * lax.rsqrt(var + 1e-5)
        a = jnp.broadcast_to(t[:, None, :], (NUM_VALS, NUM_VALS, EMBED_DIM))
        b = jnp.broadcast_to(t[None, :, :], (NUM_VALS, NUM_VALS, EMBED_DIM))
        o_ref[...] = jnp.concatenate([a, b], axis=-1).reshape(
            NPAIR, 2 * EMBED_DIM
        )

    return pl.pallas_call(
        body,
        out_shape=jax.ShapeDtypeStruct((NPAIR, 2 * EMBED_DIM), jnp.float32),
    )(card, rank52, suit52)


@functools.lru_cache(maxsize=None)
def _make_gather_sc(n_tok: int):
    assert n_tok % (2 * NW) == 0
    tok_per_w = n_tok // NW
    np_w = tok_per_w // 2
    chunk = 128
    while np_w % chunk:
        chunk //= 2
    n_chunks = np_w // chunk
    nbuf = min(3, n_chunks)

    mesh = plsc.VectorSubcoreMesh(
        core_axis_name="c", subcore_axis_name="s", num_cores=NC, num_subcores=NS
    )

    @functools.partial(
        pl.kernel,
        mesh=mesh,
        out_type=jax.ShapeDtypeStruct((n_tok // 2, 2 * EMBED_DIM), jnp.float32),
        scratch_types=[
            pltpu.VMEM((tok_per_w,), jnp.int32),
            pltpu.VMEM((np_w,), jnp.int32),
            [pltpu.VMEM((chunk, 2 * EMBED_DIM), jnp.float32) for _ in range(nbuf)],
            pltpu.VMEM_SHARED((NPAIR, 2 * EMBED_DIM), jnp.float32),
            [pltpu.SemaphoreType.DMA for _ in range(nbuf)],
            [pltpu.SemaphoreType.DMA for _ in range(nbuf)],
        ],
        compiler_params=pltpu.CompilerParams(
            use_tc_tiling_on_sc=False, needs_layout_passes=False
        ),
    )
    def gather(tab2_hbm, x_hbm, out_hbm, xloc, xp, rows, tab2_sh, gsems, ssems):
        sid = lax.axis_index("s")
        wid = sid * NC + lax.axis_index("c")
        base = wid * np_w

        @pl.when(sid == 0)
        def _():
            pltpu.sync_copy(tab2_hbm, tab2_sh)

        pltpu.sync_copy(x_hbm.at[pl.ds(wid * tok_per_w, tok_per_w)], xloc)

        i2 = lax.iota(jnp.int32, L) * 2

        def mk_idx(i, carry):
            base_t = i * (2 * L)
            ev = plsc.load_gather(xloc, [i2 + base_t])
            od = plsc.load_gather(xloc, [i2 + (base_t + 1)])
            xp[pl.ds(i * L, L)] = ev * NUM_VALS + od
            return carry

        lax.fori_loop(0, np_w // L, mk_idx, 0)

        plsc.subcore_barrier()

        def start_gather(c):
            b = c % nbuf
            return pltpu.async_copy(
                tab2_sh.at[xp.at[pl.ds(c * chunk, chunk)]], rows[b], gsems[b]
            )

        def start_scatter(c):
            b = c % nbuf
            return pltpu.async_copy(
                rows[b], out_hbm.at[pl.ds(base + c * chunk, chunk)], ssems[b]
            )

        g_h = {0: start_gather(0)}
        s_h = {}
        s_waited = set()
        for c in range(n_chunks):
            g_h[c].wait()
            if c + 1 < n_chunks:
                prev = c + 1 - nbuf
                if prev >= 0:
                    s_h[prev].wait()
                    s_waited.add(prev)
                g_h[c + 1] = start_gather(c + 1)
            s_h[c] = start_scatter(c)
        for c in range(n_chunks):
            if c not in s_waited:
                s_h[c].wait()

    return gather


@functools.lru_cache(maxsize=None)
def _make_transpose_tc(bn: int, num_cards: int):
    B = 512
    assert bn % B == 0

    def body(i_ref, o_ref):
        for c2 in range(num_cards // 2):
            sub = i_ref[:, c2, :]
            for h in range(2):
                o_ref[2 * c2 + h] = sub[:, h * EMBED_DIM:(h + 1) * EMBED_DIM].T

    return pl.pallas_call(
        body,
        grid=(bn // B,),
        in_specs=[
            pl.BlockSpec((B, num_cards // 2, 2 * EMBED_DIM), lambda i: (i, 0, 0))
        ],
        out_specs=pl.BlockSpec((num_cards, EMBED_DIM, B), lambda i: (0, 0, i)),
        out_shape=jax.ShapeDtypeStruct(
            (num_cards, EMBED_DIM, bn), jnp.float32
        ),
    )


def kernel(x, card, rank, suit):
    bn, num_cards = x.shape
    rank52 = jnp.tile(rank, (NUM_VALS // 13, 1))
    suit52 = jnp.repeat(suit, 13, axis=0)
    table2 = _pair_table_tc(card, rank52, suit52)
    out2 = _make_gather_sc(bn * num_cards)(table2, x.reshape(-1))
    out3 = out2.reshape(bn, num_cards // 2, 2 * EMBED_DIM)
    out_t = _make_transpose_tc(bn, num_cards)(out3)
    return jnp.transpose(out_t, (2, 0, 1))


# speedup vs baseline: 3.4191x; 1.5427x over previous
import functools

import jax
import jax.numpy as jnp
from jax import lax
from jax.experimental import pallas as pl
from jax.experimental.pallas import tpu as pltpu
from jax.experimental.pallas import tpu_sc as plsc

EMBED_DIM = 64
NUM_VALS = 52
NPAIR = NUM_VALS * NUM_VALS
NC = 2
NS = 16
NW = NC * NS
L = 16


def _pair_table_tc(card, rank52, suit52):
    def body(c_ref, r_ref, s_ref, o_ref):
        h = c_ref[...] + r_ref[...] + s_ref[...]
        mean = jnp.mean(h, axis=-1, keepdims=True)
        var = jnp.mean(jnp.square(h - mean), axis=-1, keepdims=True)
        t = (h - mean) * lax.rsqrt(var + 1e-5)
        a = jnp.broadcast_to(t[:, None, :], (NUM_VALS, NUM_VALS, EMBED_DIM))
        b = jnp.broadcast_to(t[None, :, :], (NUM_VALS, NUM_VALS, EMBED_DIM))
        o_ref[...] = jnp.concatenate([a, b], axis=-1).reshape(
            NPAIR, 2 * EMBED_DIM
        )

    return pl.pallas_call(
        body,
        out_shape=jax.ShapeDtypeStruct((NPAIR, 2 * EMBED_DIM), jnp.float32),
    )(card, rank52, suit52)


@functools.lru_cache(maxsize=None)
def _make_gather_sc(bn: int, num_cards: int):
    n_tok = bn * num_cards
    assert n_tok % (2 * NW) == 0
    tok_per_w = n_tok // NW
    np_w = tok_per_w // 2
    b_per_w = tok_per_w // num_cards
    chunk = 128
    while b_per_w % chunk:
        chunk //= 2
    n_chunks = np_w // chunk
    nbuf = min(3, n_chunks)

    mesh = plsc.VectorSubcoreMesh(
        core_axis_name="c", subcore_axis_name="s", num_cores=NC, num_subcores=NS
    )

    @functools.partial(
        pl.kernel,
        mesh=mesh,
        out_type=jax.ShapeDtypeStruct((n_tok // 2, 2 * EMBED_DIM), jnp.float32),
        scratch_types=[
            pltpu.VMEM((tok_per_w,), jnp.int32),
            pltpu.VMEM((np_w,), jnp.int32),
            [pltpu.VMEM((chunk, 2 * EMBED_DIM), jnp.float32) for _ in range(nbuf)],
            pltpu.VMEM_SHARED((NPAIR, 2 * EMBED_DIM), jnp.float32),
            [pltpu.SemaphoreType.DMA for _ in range(nbuf)],
            [pltpu.SemaphoreType.DMA for _ in range(nbuf)],
        ],
        compiler_params=pltpu.CompilerParams(
            use_tc_tiling_on_sc=False, needs_layout_passes=False
        ),
    )
    def gather(tab2_hbm, x_hbm, out_hbm, xloc, xp, rows, tab2_sh, gsems, ssems):
        sid = lax.axis_index("s")
        wid = sid * NC + lax.axis_index("c")
        base_b = wid * b_per_w

        @pl.when(sid == 0)
        def _():
            pltpu.sync_copy(tab2_hbm, tab2_sh)

        pltpu.sync_copy(x_hbm.at[pl.ds(wid * tok_per_w, tok_per_w)], xloc)

        ib = lax.iota(jnp.int32, L) * num_cards
        groups_per_c2 = b_per_w // L

        def mk_idx(i, carry):
            c2 = i // groups_per_c2
            j = i % groups_per_c2
            vec = ib + (j * L * num_cards + 2 * c2)
            ev = plsc.load_gather(xloc, [vec])
            od = plsc.load_gather(xloc, [vec + 1])
            xp[pl.ds(i * L, L)] = ev * NUM_VALS + od
            return carry

        lax.fori_loop(0, np_w // L, mk_idx, 0)

        plsc.subcore_barrier()

        def start_gather(c):
            b = c % nbuf
            return pltpu.async_copy(
                tab2_sh.at[xp.at[pl.ds(c * chunk, chunk)]], rows[b], gsems[b]
            )

        def start_scatter(c):
            b = c % nbuf
            c2 = (c * chunk) // b_per_w
            db0 = (c * chunk) % b_per_w
            return pltpu.async_copy(
                rows[b],
                out_hbm.at[pl.ds(c2 * bn + base_b + db0, chunk)],
                ssems[b],
            )

        g_h = {0: start_gather(0)}
        s_h = {}
        s_waited = set()
        for c in range(n_chunks):
            g_h[c].wait()
            if c + 1 < n_chunks:
                prev = c + 1 - nbuf
                if prev >= 0:
                    s_h[prev].wait()
                    s_waited.add(prev)
                g_h[c + 1] = start_gather(c + 1)
            s_h[c] = start_scatter(c)
        for c in range(n_chunks):
            if c not in s_waited:
                s_h[c].wait()

    return gather


@functools.lru_cache(maxsize=None)
def _make_transpose_tc(bn: int, num_cards: int):
    B = 2048
    while bn % B:
        B //= 2

    def body(i_ref, o_ref):
        sub = i_ref[0]
        for h in range(2):
            o_ref[h] = sub[:, h * EMBED_DIM:(h + 1) * EMBED_DIM].T

    return pl.pallas_call(
        body,
        grid=(num_cards // 2, bn // B),
        in_specs=[pl.BlockSpec((1, B, 2 * EMBED_DIM), lambda c2, i: (c2, i, 0))],
        out_specs=pl.BlockSpec((2, EMBED_DIM, B), lambda c2, i: (c2, 0, i)),
        out_shape=jax.ShapeDtypeStruct(
            (num_cards, EMBED_DIM, bn), jnp.float32
        ),
    )


def kernel(x, card, rank, suit):
    bn, num_cards = x.shape
    rank52 = jnp.tile(rank, (NUM_VALS // 13, 1))
    suit52 = jnp.repeat(suit, 13, axis=0)
    table2 = _pair_table_tc(card, rank52, suit52)
    out2 = _make_gather_sc(bn, num_cards)(table2, x.reshape(-1))
    out3 = out2.reshape(num_cards // 2, bn, 2 * EMBED_DIM)
    out_t = _make_transpose_tc(bn, num_cards)(out3)
    return jnp.transpose(out_t, (2, 0, 1))
